# Initial kernel scaffold; baseline (speedup 1.0000x reference)
import jax
import jax.numpy as jnp
from jax.experimental import pallas as pl


def kernel(hidden_states, phi_W, psi_W, gat1_W, gat1_att, gat2_W, gat2_att, pool_W, proj_W1, proj_b1, proj_W2, proj_b2):
    raise NotImplementedError("write your pallas kernel here")


# speedup vs baseline: 17.0312x; 17.0312x over previous
import functools

import jax
import jax.numpy as jnp
from jax.experimental import pallas as pl
from jax.experimental.pallas import tpu as pltpu

B, S, D = 2, 2048, 768
HEADS = 4
K = 8
SEM = 512
N = B * S

TI = 256
NI = S // TI
TR = 512

_f32 = jnp.float32


def _dot(a, b, ca, cb):
    return jax.lax.dot_general(
        a, b, (((ca,), (cb,)), ((), ())), preferred_element_type=_f32)


def _leaky(x):
    return jnp.where(x >= 0, x, 0.2 * x)


def _proj_kernel(x_ref, pw_ref, sw_ref, ph_ref, sh_ref):
    x = x_ref[...]
    ph_ref[...] = _dot(x, pw_ref[...], 1, 1)
    sh_ref[...] = _dot(x, sw_ref[...], 1, 1)


def _run_proj(x, phi_W, psi_W):
    return pl.pallas_call(
        _proj_kernel,
        grid=(N // TR,),
        in_specs=[
            pl.BlockSpec((TR, D), lambda i: (i, 0)),
            pl.BlockSpec((D, D), lambda i: (0, 0)),
            pl.BlockSpec((D, D), lambda i: (0, 0)),
        ],
        out_specs=[
            pl.BlockSpec((TR, D), lambda i: (i, 0)),
            pl.BlockSpec((TR, D), lambda i: (i, 0)),
        ],
        out_shape=[
            jax.ShapeDtypeStruct((N, D), _f32),
            jax.ShapeDtypeStruct((N, D), _f32),
        ],
    )(x, phi_W, psi_W)


def _topk_kernel(ph_ref, sh_ref, idx_ref, pm_ref):
    i = pl.program_id(1)
    phi = ph_ref[0]
    psi = sh_ref[0]
    aff = jnp.exp(_dot(phi, psi, 1, 1))
    iota = jax.lax.broadcasted_iota(jnp.int32, (TI, S), 1)
    work = aff
    vals = []
    idxs = []
    for _ in range(K):
        m = jnp.max(work, axis=1, keepdims=True)
        amin = jnp.min(jnp.where(work == m, iota, S), axis=1,
                       keepdims=True)
        vals.append(m)
        idxs.append(amin)
        work = jnp.where(iota == amin, -1.0, work)
    v = jnp.concatenate(vals, axis=1)
    ix = jnp.concatenate(idxs, axis=1)
    w = v / (jnp.sum(v, axis=1, keepdims=True) + 1e-8)
    row = i * TI + jax.lax.broadcasted_iota(jnp.int32, (TI, K), 0)
    mask = (ix != row).astype(_f32)
    pm_ref[0] = jnp.clip(w, 1e-8, None) * mask
    idx_ref[0] = ix


def _run_topk(phi_h, psi_h):
    ph = phi_h.reshape(B, S, D)
    sh = psi_h.reshape(B, S, D)
    return pl.pallas_call(
        _topk_kernel,
        grid=(B, NI),
        in_specs=[
            pl.BlockSpec((1, TI, D), lambda b, i: (b, i, 0)),
            pl.BlockSpec((1, S, D), lambda b, i: (b, 0, 0)),
        ],
        out_specs=[
            pl.BlockSpec((1, TI, K), lambda b, i: (b, i, 0)),
            pl.BlockSpec((1, TI, K), lambda b, i: (b, i, 0)),
        ],
        out_shape=[
            jax.ShapeDtypeStruct((B, S, K), jnp.int32),
            jax.ShapeDtypeStruct((B, S, K), _f32),
        ],
    )(ph, sh)


def _gat_proj_kernel(x_ref, w_ref, as_ref, ad_ref, h_ref, ab_ref):
    hflat = _dot(x_ref[...], w_ref[...], 1, 1)
    h3 = hflat.reshape(TR, HEADS, D)
    a = jnp.sum(h3 * as_ref[...][None], axis=2)
    b = jnp.sum(h3 * ad_ref[...][None], axis=2)
    for hh in range(HEADS):
        h_ref[hh] = h3[:, hh, :]
    ab_ref[...] = jnp.concatenate([a, b], axis=1)


def _run_gat_proj(x, W, att_s, att_d):
    return pl.pallas_call(
        _gat_proj_kernel,
        grid=(N // TR,),
        in_specs=[
            pl.BlockSpec((TR, D), lambda i: (i, 0)),
            pl.BlockSpec((HEADS * D, D), lambda i: (0, 0)),
            pl.BlockSpec((HEADS, D), lambda i: (0, 0)),
            pl.BlockSpec((HEADS, D), lambda i: (0, 0)),
        ],
        out_specs=[
            pl.BlockSpec((HEADS, TR, D), lambda i: (0, i, 0)),
            pl.BlockSpec((TR, 2 * HEADS), lambda i: (i, 0)),
        ],
        out_shape=[
            jax.ShapeDtypeStruct((HEADS, N, D), _f32),
            jax.ShapeDtypeStruct((N, 2 * HEADS), _f32),
        ],
    )(x, W, att_s, att_d)


def _agg_kernel(h_ref, ab_ref, abf_ref, idx_ref, pm_ref, out_ref, attn_ref):
    hid = pl.program_id(1)
    i = pl.program_id(2)

    @pl.when(i == 0)
    def _init():
        out_ref[...] = jnp.zeros_like(out_ref)
        attn_ref[...] = jnp.zeros_like(attn_ref)

    htile = h_ref[0]
    idx = idx_ref[0]
    pm = pm_ref[0]

    sel_a = (jax.lax.broadcasted_iota(jnp.int32, (1, 2 * HEADS), 1)
             == hid).astype(_f32)
    sel_b = (jax.lax.broadcasted_iota(jnp.int32, (1, 2 * HEADS), 1)
             == hid + HEADS).astype(_f32)
    a_h = jnp.sum(ab_ref[...] * sel_a, axis=1, keepdims=True)
    bf_h = jnp.sum(abf_ref[...] * sel_b, axis=1, keepdims=True)

    iota = jax.lax.broadcasted_iota(jnp.int32, (TI, S), 1)
    E = jnp.zeros((TI, S), _f32)
    for k in range(K):
        mk = (iota == idx[:, k][:, None]).astype(_f32)
        bg = _dot(mk, bf_h, 1, 0)
        wk = pm[:, k][:, None] * jnp.exp(_leaky(a_h + bg))
        E = E + mk * wk

    out_ref[0, 0] = out_ref[0, 0] + _dot(E, htile, 0, 0)
    attn_ref[0, 0] = attn_ref[0, 0] + jnp.sum(E, axis=0, keepdims=True)


def _run_agg(h, ab, idx, pm):
    return pl.pallas_call(
        _agg_kernel,
        grid=(B, HEADS, NI),
        in_specs=[
            pl.BlockSpec((1, TI, D), lambda b, h, i: (h, b * NI + i, 0)),
            pl.BlockSpec((TI, 2 * HEADS), lambda b, h, i: (b * NI + i, 0)),
            pl.BlockSpec((S, 2 * HEADS), lambda b, h, i: (b, 0)),
            pl.BlockSpec((1, TI, K), lambda b, h, i: (b, i, 0)),
            pl.BlockSpec((1, TI, K), lambda b, h, i: (b, i, 0)),
        ],
        out_specs=[
            pl.BlockSpec((1, 1, S, D), lambda b, h, i: (b, h, 0, 0)),
            pl.BlockSpec((1, 1, 1, S), lambda b, h, i: (b, h, 0, 0)),
        ],
        out_shape=[
            jax.ShapeDtypeStruct((B, HEADS, S, D), _f32),
            jax.ShapeDtypeStruct((B, HEADS, 1, S), _f32),
        ],
    )(h, ab, ab, idx, pm)


TS = 512


def _norm_kernel(out_ref, attn_ref, x_ref):
    o = out_ref[0]
    at = attn_ref[0, :, 0, :]
    xn = jnp.mean(o / (at[:, :, None] + 1e-8), axis=0)
    x_ref[0] = jnp.maximum(xn, 0.0)


def _run_norm(out, attn):
    return pl.pallas_call(
        _norm_kernel,
        grid=(B, S // TS),
        in_specs=[
            pl.BlockSpec((1, HEADS, TS, D), lambda b, j: (b, 0, j, 0)),
            pl.BlockSpec((1, HEADS, 1, TS), lambda b, j: (b, 0, 0, j)),
        ],
        out_specs=pl.BlockSpec((1, TS, D), lambda b, j: (b, j, 0)),
        out_shape=jax.ShapeDtypeStruct((B, S, D), _f32),
    )(out, attn)


def _head_kernel(x_ref, pw_ref, w1_ref, b1_ref, w2_ref, b2_ref, o_ref):
    pooled = []
    for b in range(B):
        xb = x_ref[b]
        p = _dot(xb, pw_ref[...], 1, 1)
        p = p - jnp.max(p, axis=0, keepdims=True)
        al = jnp.exp(p)
        al = al / jnp.sum(al, axis=0, keepdims=True)
        pooled.append(_dot(al, xb, 0, 0))
    pooled = jnp.concatenate(pooled, axis=0)
    hmid = jnp.maximum(_dot(pooled, w1_ref[...], 1, 1) + b1_ref[...], 0.0)
    o_ref[...] = _dot(hmid, w2_ref[...], 1, 1) + b2_ref[...]


def _run_head(x, pool_W, W1, b1, W2, b2):
    return pl.pallas_call(
        _head_kernel,
        grid=(1,),
        in_specs=[
            pl.BlockSpec((B, S, D), lambda i: (0, 0, 0)),
            pl.BlockSpec((1, D), lambda i: (0, 0)),
            pl.BlockSpec((D // 2, D), lambda i: (0, 0)),
            pl.BlockSpec((1, D // 2), lambda i: (0, 0)),
            pl.BlockSpec((SEM, D // 2), lambda i: (0, 0)),
            pl.BlockSpec((1, SEM), lambda i: (0, 0)),
        ],
        out_specs=pl.BlockSpec((B, SEM), lambda i: (0, 0)),
        out_shape=jax.ShapeDtypeStruct((B, SEM), _f32),
    )(x, pool_W, W1, b1.reshape(1, -1), W2, b2.reshape(1, -1))


@jax.jit
def kernel(hidden_states, phi_W, psi_W, gat1_W, gat1_att, gat2_W, gat2_att,
           pool_W, proj_W1, proj_b1, proj_W2, proj_b2):
    x = hidden_states.reshape(N, D)
    phi_h, psi_h = _run_proj(x, phi_W, psi_W)
    idx, pm = _run_topk(phi_h, psi_h)

    for (W, att) in ((gat1_W, gat1_att), (gat2_W, gat2_att)):
        att_s = att[0, :, :D]
        att_d = att[0, :, D:]
        h, ab = _run_gat_proj(x, W, att_s, att_d)
        out, attn = _run_agg(h, ab, idx, pm)
        xb = _run_norm(out, attn)
        x = xb.reshape(N, D)

    return _run_head(xb, pool_W, proj_W1, proj_b1, proj_W2, proj_b2)


# speedup vs baseline: 23.7801x; 1.3963x over previous
import functools

import jax
import jax.numpy as jnp
from jax.experimental import pallas as pl
from jax.experimental.pallas import tpu as pltpu

B, S, D = 2, 2048, 768
HEADS = 4
K = 8
SEM = 512
N = B * S

TI = 256
NI = S // TI
TR = 512

_f32 = jnp.float32


def _dot(a, b, ca, cb):
    return jax.lax.dot_general(
        a, b, (((ca,), (cb,)), ((), ())), preferred_element_type=_f32)


def _leaky(x):
    return jnp.where(x >= 0, x, 0.2 * x)


def _proj_kernel(x_ref, pw_ref, sw_ref, ph_ref, sh_ref):
    x = x_ref[...]
    ph_ref[...] = _dot(x, pw_ref[...], 1, 1)
    sh_ref[...] = _dot(x, sw_ref[...], 1, 1)


def _run_proj(x, phi_W, psi_W):
    return pl.pallas_call(
        _proj_kernel,
        grid=(N // TR,),
        in_specs=[
            pl.BlockSpec((TR, D), lambda i: (i, 0)),
            pl.BlockSpec((D, D), lambda i: (0, 0)),
            pl.BlockSpec((D, D), lambda i: (0, 0)),
        ],
        out_specs=[
            pl.BlockSpec((TR, D), lambda i: (i, 0)),
            pl.BlockSpec((TR, D), lambda i: (i, 0)),
        ],
        out_shape=[
            jax.ShapeDtypeStruct((N, D), _f32),
            jax.ShapeDtypeStruct((N, D), _f32),
        ],
    )(x, phi_W, psi_W)


def _topk_kernel(ph_ref, sh_ref, idx_ref, pm_ref):
    i = pl.program_id(1)
    phi = ph_ref[0]
    psi = sh_ref[0]
    aff = jnp.exp(_dot(phi, psi, 1, 1))
    iota = jax.lax.broadcasted_iota(jnp.int32, (TI, S), 1)
    work = aff
    vals = []
    idxs = []
    for _ in range(K):
        m = jnp.max(work, axis=1, keepdims=True)
        amin = jnp.min(jnp.where(work == m, iota, S), axis=1,
                       keepdims=True)
        vals.append(m)
        idxs.append(amin)
        work = jnp.where(iota == amin, -1.0, work)
    v = jnp.concatenate(vals, axis=1)
    ix = jnp.concatenate(idxs, axis=1)
    w = v / (jnp.sum(v, axis=1, keepdims=True) + 1e-8)
    row = i * TI + jax.lax.broadcasted_iota(jnp.int32, (TI, K), 0)
    mask = (ix != row).astype(_f32)
    pm_ref[0] = jnp.clip(w, 1e-8, None) * mask
    idx_ref[0] = ix


def _run_topk(phi_h, psi_h):
    ph = phi_h.reshape(B, S, D)
    sh = psi_h.reshape(B, S, D)
    return pl.pallas_call(
        _topk_kernel,
        grid=(B, NI),
        in_specs=[
            pl.BlockSpec((1, TI, D), lambda b, i: (b, i, 0)),
            pl.BlockSpec((1, S, D), lambda b, i: (b, 0, 0)),
        ],
        out_specs=[
            pl.BlockSpec((1, TI, K), lambda b, i: (b, i, 0)),
            pl.BlockSpec((1, TI, K), lambda b, i: (b, i, 0)),
        ],
        out_shape=[
            jax.ShapeDtypeStruct((B, S, K), jnp.int32),
            jax.ShapeDtypeStruct((B, S, K), _f32),
        ],
    )(ph, sh)


def _gat_proj_kernel(x_ref, w_ref, as_ref, ad_ref, h_ref, ab_ref):
    hflat = _dot(x_ref[...], w_ref[...], 1, 1)
    h3 = hflat.reshape(TR, HEADS, D)
    a = jnp.sum(h3 * as_ref[...][None], axis=2)
    b = jnp.sum(h3 * ad_ref[...][None], axis=2)
    for hh in range(HEADS):
        h_ref[hh] = h3[:, hh, :]
    ab_ref[...] = jnp.concatenate([a, b], axis=1)


def _run_gat_proj(x, W, att_s, att_d):
    return pl.pallas_call(
        _gat_proj_kernel,
        grid=(N // TR,),
        in_specs=[
            pl.BlockSpec((TR, D), lambda i: (i, 0)),
            pl.BlockSpec((HEADS * D, D), lambda i: (0, 0)),
            pl.BlockSpec((HEADS, D), lambda i: (0, 0)),
            pl.BlockSpec((HEADS, D), lambda i: (0, 0)),
        ],
        out_specs=[
            pl.BlockSpec((HEADS, TR, D), lambda i: (0, i, 0)),
            pl.BlockSpec((TR, 2 * HEADS), lambda i: (i, 0)),
        ],
        out_shape=[
            jax.ShapeDtypeStruct((HEADS, N, D), _f32),
            jax.ShapeDtypeStruct((N, 2 * HEADS), _f32),
        ],
    )(x, W, att_s, att_d)


_bf16 = jnp.bfloat16


def _edgew_kernel(ab_ref, abf_ref, idx_ref, pm_ref, w_ref):
    a = ab_ref[...][:, :HEADS]
    bfull = abf_ref[...][:, HEADS:]
    idx = idx_ref[0]
    pm = pm_ref[0]
    iota = jax.lax.broadcasted_iota(jnp.int32, (TI, S), 1)
    cols = []
    for k in range(K):
        mk = (iota == idx[:, k][:, None]).astype(_f32)
        bg = _dot(mk, bfull, 1, 0)
        cols.append(pm[:, k][:, None] * jnp.exp(_leaky(a + bg)))
    w_ref[0] = jnp.concatenate(cols, axis=1)


def _run_edgew(ab, idx, pm):
    return pl.pallas_call(
        _edgew_kernel,
        grid=(B, NI),
        in_specs=[
            pl.BlockSpec((TI, 2 * HEADS), lambda b, i: (b * NI + i, 0)),
            pl.BlockSpec((S, 2 * HEADS), lambda b, i: (b, 0)),
            pl.BlockSpec((1, TI, K), lambda b, i: (b, i, 0)),
            pl.BlockSpec((1, TI, K), lambda b, i: (b, i, 0)),
        ],
        out_specs=pl.BlockSpec((1, TI, K * HEADS), lambda b, i: (b, i, 0)),
        out_shape=jax.ShapeDtypeStruct((B, S, K * HEADS), _f32),
    )(ab, ab, idx, pm)


def _agg_kernel(h_ref, w_ref, idx_ref, out_ref, attn_ref):
    hid = pl.program_id(1)
    i = pl.program_id(2)

    @pl.when(i == 0)
    def _init():
        out_ref[...] = jnp.zeros_like(out_ref)
        attn_ref[...] = jnp.zeros_like(attn_ref)

    htile = h_ref[0]
    wt = w_ref[0]
    idx16 = idx_ref[0].astype(jnp.int16)

    iota = jax.lax.broadcasted_iota(jnp.int16, (TI, S), 1)
    cid = jax.lax.broadcasted_iota(jnp.int32, (1, K * HEADS), 1)
    E = jnp.zeros((TI, S), _bf16)
    for k in range(K):
        match = iota == idx16[:, k][:, None]
        sel = (cid == k * HEADS + hid).astype(_f32)
        wk = jnp.sum(wt * sel, axis=1, keepdims=True)
        E = jnp.where(match, wk.astype(_bf16), E)

    x = jnp.concatenate(
        [htile.astype(_bf16), jnp.ones((TI, 128), _bf16)], axis=1)
    acc = _dot(E, x, 0, 0)
    out_ref[0, 0] = out_ref[0, 0] + acc[:, :D]
    attn_ref[0, 0] = attn_ref[0, 0] + acc[:, D:D + 1]


def _run_agg(h, w, idx):
    return pl.pallas_call(
        _agg_kernel,
        grid=(B, HEADS, NI),
        in_specs=[
            pl.BlockSpec((1, TI, D), lambda b, h, i: (h, b * NI + i, 0)),
            pl.BlockSpec((1, TI, K * HEADS), lambda b, h, i: (b, i, 0)),
            pl.BlockSpec((1, TI, K), lambda b, h, i: (b, i, 0)),
        ],
        out_specs=[
            pl.BlockSpec((1, 1, S, D), lambda b, h, i: (b, h, 0, 0)),
            pl.BlockSpec((1, 1, S, 1), lambda b, h, i: (b, h, 0, 0)),
        ],
        out_shape=[
            jax.ShapeDtypeStruct((B, HEADS, S, D), _f32),
            jax.ShapeDtypeStruct((B, HEADS, S, 1), _f32),
        ],
    )(h, w, idx)


TS = 512


def _norm_kernel(out_ref, attn_ref, x_ref):
    o = out_ref[0]
    at = attn_ref[0, :, :, 0]
    xn = jnp.mean(o / (at[:, :, None] + 1e-8), axis=0)
    x_ref[0] = jnp.maximum(xn, 0.0)


def _run_norm(out, attn):
    return pl.pallas_call(
        _norm_kernel,
        grid=(B, S // TS),
        in_specs=[
            pl.BlockSpec((1, HEADS, TS, D), lambda b, j: (b, 0, j, 0)),
            pl.BlockSpec((1, HEADS, TS, 1), lambda b, j: (b, 0, j, 0)),
        ],
        out_specs=pl.BlockSpec((1, TS, D), lambda b, j: (b, j, 0)),
        out_shape=jax.ShapeDtypeStruct((B, S, D), _f32),
    )(out, attn)


def _head_kernel(x_ref, pw_ref, w1_ref, b1_ref, w2_ref, b2_ref, o_ref):
    pooled = []
    for b in range(B):
        xb = x_ref[b]
        p = _dot(xb, pw_ref[...], 1, 1)
        p = p - jnp.max(p, axis=0, keepdims=True)
        al = jnp.exp(p)
        al = al / jnp.sum(al, axis=0, keepdims=True)
        pooled.append(_dot(al, xb, 0, 0))
    pooled = jnp.concatenate(pooled, axis=0)
    hmid = jnp.maximum(_dot(pooled, w1_ref[...], 1, 1) + b1_ref[...], 0.0)
    o_ref[...] = _dot(hmid, w2_ref[...], 1, 1) + b2_ref[...]


def _run_head(x, pool_W, W1, b1, W2, b2):
    return pl.pallas_call(
        _head_kernel,
        grid=(1,),
        in_specs=[
            pl.BlockSpec((B, S, D), lambda i: (0, 0, 0)),
            pl.BlockSpec((1, D), lambda i: (0, 0)),
            pl.BlockSpec((D // 2, D), lambda i: (0, 0)),
            pl.BlockSpec((1, D // 2), lambda i: (0, 0)),
            pl.BlockSpec((SEM, D // 2), lambda i: (0, 0)),
            pl.BlockSpec((1, SEM), lambda i: (0, 0)),
        ],
        out_specs=pl.BlockSpec((B, SEM), lambda i: (0, 0)),
        out_shape=jax.ShapeDtypeStruct((B, SEM), _f32),
    )(x, pool_W, W1, b1.reshape(1, -1), W2, b2.reshape(1, -1))


@jax.jit
def kernel(hidden_states, phi_W, psi_W, gat1_W, gat1_att, gat2_W, gat2_att,
           pool_W, proj_W1, proj_b1, proj_W2, proj_b2):
    x = hidden_states.reshape(N, D)
    phi_h, psi_h = _run_proj(x, phi_W, psi_W)
    idx, pm = _run_topk(phi_h, psi_h)

    for (W, att) in ((gat1_W, gat1_att), (gat2_W, gat2_att)):
        att_s = att[0, :, :D]
        att_d = att[0, :, D:]
        h, ab = _run_gat_proj(x, W, att_s, att_d)
        w = _run_edgew(ab, idx, pm)
        out, attn = _run_agg(h, w, idx)
        xb = _run_norm(out, attn)
        x = xb.reshape(N, D)

    return _run_head(xb, pool_W, proj_W1, proj_b1, proj_W2, proj_b2)


# speedup vs baseline: 25.9299x; 1.0904x over previous
import functools

import jax
import jax.numpy as jnp
from jax.experimental import pallas as pl
from jax.experimental.pallas import tpu as pltpu

B, S, D = 2, 2048, 768
HEADS = 4
K = 8
SEM = 512
N = B * S

TI = 256
NI = S // TI
TR = 512

_f32 = jnp.float32


def _dot(a, b, ca, cb):
    return jax.lax.dot_general(
        a, b, (((ca,), (cb,)), ((), ())), preferred_element_type=_f32)


def _leaky(x):
    return jnp.where(x >= 0, x, 0.2 * x)


def _proj_kernel(x_ref, pw_ref, sw_ref, ph_ref, sh_ref):
    x = x_ref[...]
    ph_ref[...] = _dot(x, pw_ref[...], 1, 1)
    sh_ref[...] = _dot(x, sw_ref[...], 1, 1)


def _run_proj(x, phi_W, psi_W):
    return pl.pallas_call(
        _proj_kernel,
        grid=(N // TR,),
        in_specs=[
            pl.BlockSpec((TR, D), lambda i: (i, 0)),
            pl.BlockSpec((D, D), lambda i: (0, 0)),
            pl.BlockSpec((D, D), lambda i: (0, 0)),
        ],
        out_specs=[
            pl.BlockSpec((TR, D), lambda i: (i, 0)),
            pl.BlockSpec((TR, D), lambda i: (i, 0)),
        ],
        out_shape=[
            jax.ShapeDtypeStruct((N, D), _f32),
            jax.ShapeDtypeStruct((N, D), _f32),
        ],
    )(x, phi_W, psi_W)


def _topk_kernel(ph_ref, sh_ref, idx_ref, pm_ref):
    i = pl.program_id(1)
    phi = ph_ref[0]
    psi = sh_ref[0]
    aff = jnp.exp(_dot(phi, psi, 1, 1))
    iota = jax.lax.broadcasted_iota(jnp.int32, (TI, S), 1)
    work = aff
    vals = []
    idxs = []
    for _ in range(K):
        m = jnp.max(work, axis=1, keepdims=True)
        amin = jnp.min(jnp.where(work == m, iota, S), axis=1,
                       keepdims=True)
        vals.append(m)
        idxs.append(amin)
        work = jnp.where(iota == amin, -1.0, work)
    v = jnp.concatenate(vals, axis=1)
    ix = jnp.concatenate(idxs, axis=1)
    w = v / (jnp.sum(v, axis=1, keepdims=True) + 1e-8)
    row = i * TI + jax.lax.broadcasted_iota(jnp.int32, (TI, K), 0)
    mask = (ix != row).astype(_f32)
    pm_ref[0] = jnp.clip(w, 1e-8, None) * mask
    idx_ref[0] = ix


def _run_topk(phi_h, psi_h):
    ph = phi_h.reshape(B, S, D)
    sh = psi_h.reshape(B, S, D)
    return pl.pallas_call(
        _topk_kernel,
        grid=(B, NI),
        in_specs=[
            pl.BlockSpec((1, TI, D), lambda b, i: (b, i, 0)),
            pl.BlockSpec((1, S, D), lambda b, i: (b, 0, 0)),
        ],
        out_specs=[
            pl.BlockSpec((1, TI, K), lambda b, i: (b, i, 0)),
            pl.BlockSpec((1, TI, K), lambda b, i: (b, i, 0)),
        ],
        out_shape=[
            jax.ShapeDtypeStruct((B, S, K), jnp.int32),
            jax.ShapeDtypeStruct((B, S, K), _f32),
        ],
    )(ph, sh)


def _gat_proj_kernel(x_ref, w_ref, as_ref, ad_ref, h_ref, ab_ref):
    hflat = _dot(x_ref[...], w_ref[...], 1, 1)
    h3 = hflat.reshape(TR, HEADS, D)
    a = jnp.sum(h3 * as_ref[...][None], axis=2)
    b = jnp.sum(h3 * ad_ref[...][None], axis=2)
    for hh in range(HEADS):
        h_ref[hh] = h3[:, hh, :]
    ab_ref[...] = jnp.concatenate([a, b], axis=1)


def _run_gat_proj(x, W, att_s, att_d):
    return pl.pallas_call(
        _gat_proj_kernel,
        grid=(N // TR,),
        in_specs=[
            pl.BlockSpec((TR, D), lambda i: (i, 0)),
            pl.BlockSpec((HEADS * D, D), lambda i: (0, 0)),
            pl.BlockSpec((HEADS, D), lambda i: (0, 0)),
            pl.BlockSpec((HEADS, D), lambda i: (0, 0)),
        ],
        out_specs=[
            pl.BlockSpec((HEADS, TR, D), lambda i: (0, i, 0)),
            pl.BlockSpec((TR, 2 * HEADS), lambda i: (i, 0)),
        ],
        out_shape=[
            jax.ShapeDtypeStruct((HEADS, N, D), _f32),
            jax.ShapeDtypeStruct((N, 2 * HEADS), _f32),
        ],
    )(x, W, att_s, att_d)


_bf16 = jnp.bfloat16


def _edgew_kernel(ab_ref, abf_ref, idx_ref, pm_ref, w_ref):
    a = ab_ref[...][:, :HEADS]
    bfull = abf_ref[...][:, HEADS:]
    idx = idx_ref[0]
    pm = pm_ref[0]
    iota = jax.lax.broadcasted_iota(jnp.int32, (TI, S), 1)
    cols = []
    for k in range(K):
        mk = (iota == idx[:, k][:, None]).astype(_f32)
        bg = _dot(mk, bfull, 1, 0)
        cols.append(pm[:, k][:, None] * jnp.exp(_leaky(a + bg)))
    w_ref[0] = jnp.concatenate(cols, axis=1)


def _run_edgew(ab, idx, pm):
    return pl.pallas_call(
        _edgew_kernel,
        grid=(B, NI),
        in_specs=[
            pl.BlockSpec((TI, 2 * HEADS), lambda b, i: (b * NI + i, 0)),
            pl.BlockSpec((S, 2 * HEADS), lambda b, i: (b, 0)),
            pl.BlockSpec((1, TI, K), lambda b, i: (b, i, 0)),
            pl.BlockSpec((1, TI, K), lambda b, i: (b, i, 0)),
        ],
        out_specs=pl.BlockSpec((1, TI, K * HEADS), lambda b, i: (b, i, 0)),
        out_shape=jax.ShapeDtypeStruct((B, S, K * HEADS), _f32),
    )(ab, ab, idx, pm)


def _agg_kernel(h_ref, w_ref, idx_ref, out_ref, attn_ref):
    hid = pl.program_id(1)

    htile = h_ref[0]
    wt = w_ref[0]
    idx16 = idx_ref[0].astype(jnp.int16)

    iota = jax.lax.broadcasted_iota(jnp.int16, (S, S), 1)
    cid = jax.lax.broadcasted_iota(jnp.int32, (1, K * HEADS), 1)
    E = jnp.zeros((S, S), _bf16)
    for k in range(K):
        match = iota == idx16[:, k][:, None]
        sel = (cid == k * HEADS + hid).astype(_f32)
        wk = jnp.sum(wt * sel, axis=1, keepdims=True)
        E = jnp.where(match, wk.astype(_bf16), E)

    x = jnp.concatenate(
        [htile.astype(_bf16), jnp.ones((S, 128), _bf16)], axis=1)
    acc = _dot(E, x, 0, 0)
    out_ref[0, 0] = acc[:, :D]
    attn_ref[0, 0] = acc[:, D:D + 1]


def _run_agg(h, w, idx):
    return pl.pallas_call(
        _agg_kernel,
        grid=(B, HEADS),
        in_specs=[
            pl.BlockSpec((1, S, D), lambda b, h: (h, b, 0)),
            pl.BlockSpec((1, S, K * HEADS), lambda b, h: (b, 0, 0)),
            pl.BlockSpec((1, S, K), lambda b, h: (b, 0, 0)),
        ],
        out_specs=[
            pl.BlockSpec((1, 1, S, D), lambda b, h: (b, h, 0, 0)),
            pl.BlockSpec((1, 1, S, 1), lambda b, h: (b, h, 0, 0)),
        ],
        out_shape=[
            jax.ShapeDtypeStruct((B, HEADS, S, D), _f32),
            jax.ShapeDtypeStruct((B, HEADS, S, 1), _f32),
        ],
    )(h, w, idx)


TS = 512


def _norm_kernel(out_ref, attn_ref, x_ref):
    o = out_ref[0]
    at = attn_ref[0, :, :, 0]
    xn = jnp.mean(o / (at[:, :, None] + 1e-8), axis=0)
    x_ref[0] = jnp.maximum(xn, 0.0)


def _run_norm(out, attn):
    return pl.pallas_call(
        _norm_kernel,
        grid=(B, S // TS),
        in_specs=[
            pl.BlockSpec((1, HEADS, TS, D), lambda b, j: (b, 0, j, 0)),
            pl.BlockSpec((1, HEADS, TS, 1), lambda b, j: (b, 0, j, 0)),
        ],
        out_specs=pl.BlockSpec((1, TS, D), lambda b, j: (b, j, 0)),
        out_shape=jax.ShapeDtypeStruct((B, S, D), _f32),
    )(out, attn)


def _head_kernel(x_ref, pw_ref, w1_ref, b1_ref, w2_ref, b2_ref, o_ref):
    pooled = []
    for b in range(B):
        xb = x_ref[b]
        p = _dot(xb, pw_ref[...], 1, 1)
        p = p - jnp.max(p, axis=0, keepdims=True)
        al = jnp.exp(p)
        al = al / jnp.sum(al, axis=0, keepdims=True)
        pooled.append(_dot(al, xb, 0, 0))
    pooled = jnp.concatenate(pooled, axis=0)
    hmid = jnp.maximum(_dot(pooled, w1_ref[...], 1, 1) + b1_ref[...], 0.0)
    o_ref[...] = _dot(hmid, w2_ref[...], 1, 1) + b2_ref[...]


def _run_head(x, pool_W, W1, b1, W2, b2):
    return pl.pallas_call(
        _head_kernel,
        grid=(1,),
        in_specs=[
            pl.BlockSpec((B, S, D), lambda i: (0, 0, 0)),
            pl.BlockSpec((1, D), lambda i: (0, 0)),
            pl.BlockSpec((D // 2, D), lambda i: (0, 0)),
            pl.BlockSpec((1, D // 2), lambda i: (0, 0)),
            pl.BlockSpec((SEM, D // 2), lambda i: (0, 0)),
            pl.BlockSpec((1, SEM), lambda i: (0, 0)),
        ],
        out_specs=pl.BlockSpec((B, SEM), lambda i: (0, 0)),
        out_shape=jax.ShapeDtypeStruct((B, SEM), _f32),
    )(x, pool_W, W1, b1.reshape(1, -1), W2, b2.reshape(1, -1))


@jax.jit
def kernel(hidden_states, phi_W, psi_W, gat1_W, gat1_att, gat2_W, gat2_att,
           pool_W, proj_W1, proj_b1, proj_W2, proj_b2):
    x = hidden_states.reshape(N, D)
    phi_h, psi_h = _run_proj(x, phi_W, psi_W)
    idx, pm = _run_topk(phi_h, psi_h)

    for (W, att) in ((gat1_W, gat1_att), (gat2_W, gat2_att)):
        att_s = att[0, :, :D]
        att_d = att[0, :, D:]
        h, ab = _run_gat_proj(x, W, att_s, att_d)
        w = _run_edgew(ab, idx, pm)
        out, attn = _run_agg(h, w, idx)
        xb = _run_norm(out, attn)
        x = xb.reshape(N, D)

    return _run_head(xb, pool_W, proj_W1, proj_b1, proj_W2, proj_b2)
